# Initial kernel scaffold; baseline (speedup 1.0000x reference)
import jax
import jax.numpy as jnp
from jax.experimental import pallas as pl


def kernel(x, edge_index, W, b):
    raise NotImplementedError("write your pallas kernel here")


# speedup vs baseline: 8.5960x; 8.5960x over previous
import functools

import jax
import jax.numpy as jnp
from jax import lax
from jax.experimental import pallas as pl
from jax.experimental.pallas import tpu as pltpu
from jax.experimental.pallas import tpu_sc as plsc

N = 10000
D = 128
E = 320000
NC = 2
NS = 16
K = 125
NCHUNK = E // (NC * NS * K)
RPT = 632
RPT_LAST = N - (NS - 1) * RPT

_mesh = plsc.VectorSubcoreMesh(core_axis_name="c", subcore_axis_name="s")


@functools.partial(
    pl.kernel,
    out_type=jax.ShapeDtypeStruct((NC, N, D), jnp.float32),
    mesh=_mesh,
    scratch_types=[
        pltpu.VMEM_SHARED((N, D), jnp.float32),
        pltpu.VMEM((NCHUNK, K), jnp.int32),
        pltpu.VMEM((NCHUNK, K), jnp.int32),
        pltpu.VMEM((K, D), jnp.float32),
        pltpu.SemaphoreType.DMA,
    ],
)
def _sc_aggregate(x_hbm, src_hbm, dst_hbm, z_hbm, out_hbm,
                  acc, sidx, didx, rows, gsem):
    c = lax.axis_index("c")
    s = lax.axis_index("s")
    w = c * NS + s

    pltpu.sync_copy(src_hbm.at[w], sidx)
    pltpu.sync_copy(dst_hbm.at[w], didx)

    @pl.when(s == 0)
    def _():
        @pl.when(c == 0)
        def _():
            pltpu.sync_copy(x_hbm, acc)

        @pl.when(c != 0)
        def _():
            pltpu.sync_copy(z_hbm, acc)

    plsc.subcore_barrier()

    def body(j, carry):
        pltpu.async_copy(x_hbm.at[sidx.at[j]], rows, gsem).wait()
        pltpu.sync_copy(rows, acc.at[didx.at[j]], add=True)
        return carry

    lax.fori_loop(0, NCHUNK, body, 0)

    plsc.subcore_barrier()

    @pl.when(s < NS - 1)
    def _():
        pltpu.sync_copy(acc.at[pl.ds(s * RPT, RPT)],
                        out_hbm.at[c, pl.ds(s * RPT, RPT)])

    @pl.when(s == NS - 1)
    def _():
        pltpu.sync_copy(acc.at[pl.ds((NS - 1) * RPT, RPT_LAST)],
                        out_hbm.at[c, pl.ds((NS - 1) * RPT, RPT_LAST)])


BN = 1000


def _tc_body(p_ref, w_ref, b_ref, o_ref):
    h = p_ref[0] + p_ref[1]
    o_ref[...] = lax.dot_general(
        h, w_ref[...], (((1,), (1,)), ((), ())),
        preferred_element_type=jnp.float32) + b_ref[...]


def _tc_combine(parts, W, b2):
    return pl.pallas_call(
        _tc_body,
        grid=(N // BN,),
        in_specs=[
            pl.BlockSpec((NC, BN, D), lambda i: (0, i, 0)),
            pl.BlockSpec((D, D), lambda i: (0, 0)),
            pl.BlockSpec((1, D), lambda i: (0, 0)),
        ],
        out_specs=pl.BlockSpec((BN, D), lambda i: (i, 0)),
        out_shape=jax.ShapeDtypeStruct((N, D), jnp.float32),
    )(parts, W, b2)


def kernel(x, edge_index, W, b):
    src = edge_index[0].astype(jnp.int32).reshape(NC * NS, NCHUNK, K)
    dst = edge_index[1].astype(jnp.int32).reshape(NC * NS, NCHUNK, K)
    z = jnp.zeros_like(x)
    parts = _sc_aggregate(x, src, dst, z)
    return _tc_combine(parts, W, b.reshape(1, D))


# speedup vs baseline: 10.8863x; 1.2664x over previous
import functools

import jax
import jax.numpy as jnp
from jax import lax
from jax.experimental import pallas as pl
from jax.experimental.pallas import tpu as pltpu
from jax.experimental.pallas import tpu_sc as plsc

N = 10000
D = 128
E = 320000
NC = 2
NS = 16
NW = NC * NS
K = 125
NCHUNK = E // (NW * K)
BLK = 8
NBLK = NCHUNK // BLK
NGRP = NBLK // 2
RPT = 632
RPT_LAST = N - (NS - 1) * RPT

_mesh = plsc.VectorSubcoreMesh(core_axis_name="c", subcore_axis_name="s")


@functools.partial(
    pl.kernel,
    out_type=jax.ShapeDtypeStruct((NC, N, D), jnp.float32),
    mesh=_mesh,
    scratch_types=[
        pltpu.VMEM_SHARED((N, D), jnp.float32),
        pltpu.VMEM((2, BLK, K), jnp.int32),
        pltpu.VMEM((2, BLK, K), jnp.int32),
        pltpu.VMEM((2, K, D), jnp.float32),
        pltpu.SemaphoreType.DMA,
        pltpu.SemaphoreType.DMA,
        pltpu.SemaphoreType.DMA,
        pltpu.SemaphoreType.DMA,
        pltpu.SemaphoreType.DMA,
        pltpu.SemaphoreType.DMA,
        pltpu.SemaphoreType.DMA,
        pltpu.SemaphoreType.DMA,
    ],
)
def _sc_aggregate(x_hbm, src_hbm, dst_hbm, z_hbm, out_hbm,
                  acc, sblk, dblk, rows,
                  g0, g1, s0, s1, is0, is1, id0, id1):
    gsems = (g0, g1)
    ssems = (s0, s1)
    isems = (is0, is1)
    idsems = (id0, id1)
    c = lax.axis_index("c")
    s = lax.axis_index("s")
    w = c * NS + s

    @pl.when(s == 0)
    def _():
        @pl.when(c == 0)
        def _():
            pltpu.sync_copy(x_hbm, acc)

        @pl.when(c != 0)
        def _():
            pltpu.sync_copy(z_hbm, acc)

    pltpu.async_copy(src_hbm.at[w, 0], sblk.at[0], isems[0])
    pltpu.async_copy(dst_hbm.at[w, 0], dblk.at[0], idsems[0])
    pltpu.make_async_copy(src_hbm.at[w, 0], sblk.at[0], isems[0]).wait()
    pltpu.make_async_copy(dst_hbm.at[w, 0], dblk.at[0], idsems[0]).wait()

    plsc.subcore_barrier()

    pltpu.async_copy(x_hbm.at[sblk.at[0, 0]], rows.at[0], gsems[0])

    def body(g, carry):
        for u in range(16):
            ru, rn = u % 2, (u + 1) % 2
            p, r = u // 8, u % 8

            pltpu.make_async_copy(x_hbm.at[sblk.at[p, r]], rows.at[ru],
                                  gsems[ru]).wait()
            pltpu.async_copy(rows.at[ru], acc.at[dblk.at[p, r]], ssems[ru],
                             add=True)

            pp, rr = ((u - 1) % 16) // 8, (u - 1) % 8
            if u == 0:
                @pl.when(g >= 1)
                def _():
                    pltpu.make_async_copy(rows.at[rn], acc.at[dblk.at[1, 7]],
                                          ssems[rn]).wait()
            else:
                pltpu.make_async_copy(rows.at[rn], acc.at[dblk.at[pp, rr]],
                                      ssems[rn]).wait()

            if u == 0:
                blk = 2 * g + 1
                pltpu.async_copy(src_hbm.at[w, blk], sblk.at[1], isems[1])
                pltpu.async_copy(dst_hbm.at[w, blk], dblk.at[1], idsems[1])
            if u == 8:
                @pl.when(g < NGRP - 1)
                def _():
                    blk = 2 * g + 2
                    pltpu.async_copy(src_hbm.at[w, blk], sblk.at[0], isems[0])
                    pltpu.async_copy(dst_hbm.at[w, blk], dblk.at[0],
                                     idsems[0])
            if u == 7:
                blk = 2 * g + 1
                pltpu.make_async_copy(src_hbm.at[w, blk], sblk.at[1],
                                      isems[1]).wait()
                pltpu.make_async_copy(dst_hbm.at[w, blk], dblk.at[1],
                                      idsems[1]).wait()
            if u == 15:
                @pl.when(g < NGRP - 1)
                def _():
                    blk = 2 * g + 2
                    pltpu.make_async_copy(src_hbm.at[w, blk], sblk.at[0],
                                          isems[0]).wait()
                    pltpu.make_async_copy(dst_hbm.at[w, blk], dblk.at[0],
                                          idsems[0]).wait()

            np_, nr = ((u + 1) % 16) // 8, (u + 1) % 8
            if u == 15:
                @pl.when(g < NGRP - 1)
                def _():
                    pltpu.async_copy(x_hbm.at[sblk.at[0, 0]], rows.at[rn],
                                     gsems[rn])
            else:
                pltpu.async_copy(x_hbm.at[sblk.at[np_, nr]], rows.at[rn],
                                 gsems[rn])
        return carry

    lax.fori_loop(0, NGRP, body, 0)

    pltpu.make_async_copy(rows.at[1], acc.at[dblk.at[1, 7]], ssems[1]).wait()

    plsc.subcore_barrier()

    @pl.when(s < NS - 1)
    def _():
        pltpu.sync_copy(acc.at[pl.ds(s * RPT, RPT)],
                        out_hbm.at[c, pl.ds(s * RPT, RPT)])

    @pl.when(s == NS - 1)
    def _():
        pltpu.sync_copy(acc.at[pl.ds((NS - 1) * RPT, RPT_LAST)],
                        out_hbm.at[c, pl.ds((NS - 1) * RPT, RPT_LAST)])


BN = 1000


def _tc_body(p_ref, w_ref, b_ref, o_ref):
    h = p_ref[0] + p_ref[1]
    o_ref[...] = lax.dot_general(
        h, w_ref[...], (((1,), (1,)), ((), ())),
        preferred_element_type=jnp.float32) + b_ref[...]


def _tc_combine(parts, W, b2):
    return pl.pallas_call(
        _tc_body,
        grid=(N // BN,),
        in_specs=[
            pl.BlockSpec((NC, BN, D), lambda i: (0, i, 0)),
            pl.BlockSpec((D, D), lambda i: (0, 0)),
            pl.BlockSpec((1, D), lambda i: (0, 0)),
        ],
        out_specs=pl.BlockSpec((BN, D), lambda i: (i, 0)),
        out_shape=jax.ShapeDtypeStruct((N, D), jnp.float32),
    )(parts, W, b2)


def kernel(x, edge_index, W, b):
    src = edge_index[0].astype(jnp.int32).reshape(NW, NBLK, BLK, K)
    dst = edge_index[1].astype(jnp.int32).reshape(NW, NBLK, BLK, K)
    z = jnp.zeros_like(x)
    parts = _sc_aggregate(x, src, dst, z)
    return _tc_combine(parts, W, b.reshape(1, D))


# speedup vs baseline: 11.9015x; 1.0933x over previous
import functools

import jax
import jax.numpy as jnp
from jax import lax
from jax.experimental import pallas as pl
from jax.experimental.pallas import tpu as pltpu
from jax.experimental.pallas import tpu_sc as plsc

N = 10000
D = 128
E = 320000
NC = 2
NS = 16
NW = NC * NS
K = 125
NCHUNK = E // (NW * K)
BLK = 8
NBLK = NCHUNK // BLK
NGRP = NBLK // 2
RPT = 632
RPT_LAST = N - (NS - 1) * RPT
ZR = 40

_mesh = plsc.VectorSubcoreMesh(core_axis_name="c", subcore_axis_name="s")


@functools.partial(
    pl.kernel,
    out_type=jax.ShapeDtypeStruct((NC, N, D), jnp.float32),
    mesh=_mesh,
    scratch_types=[
        pltpu.VMEM_SHARED((N, D), jnp.float32),
        pltpu.VMEM((2, BLK, K), jnp.int32),
        pltpu.VMEM((2, BLK, K), jnp.int32),
        pltpu.VMEM((2, K, D), jnp.float32),
        pltpu.SemaphoreType.DMA,
        pltpu.SemaphoreType.DMA,
        pltpu.SemaphoreType.DMA,
        pltpu.SemaphoreType.DMA,
        pltpu.SemaphoreType.DMA,
        pltpu.SemaphoreType.DMA,
        pltpu.SemaphoreType.DMA,
        pltpu.SemaphoreType.DMA,
    ],
)
def _sc_aggregate(x_hbm, ei_hbm, out_hbm,
                  acc, sblk, dblk, rows,
                  g0, g1, s0, s1, is0, is1, id0, id1):
    gsems = (g0, g1)
    ssems = (s0, s1)
    isems = (is0, is1)
    idsems = (id0, id1)
    c = lax.axis_index("c")
    s = lax.axis_index("s")
    w = c * NS + s
    src_hbm = ei_hbm.at[0]
    dst_hbm = ei_hbm.at[1]

    pltpu.async_copy(src_hbm.at[w, 0], sblk.at[0], isems[0])
    pltpu.async_copy(dst_hbm.at[w, 0], dblk.at[0], idsems[0])

    zv = jnp.zeros((16,), jnp.float32)
    for i in range(ZR):
        for jj in range(D // 16):
            rows[0, i, pl.ds(jj * 16, 16)] = zv

    zbase = s * RPT

    @pl.when(s < NS - 1)
    def _():
        for r in range(15):
            pltpu.async_copy(rows.at[0, pl.ds(0, ZR)],
                             acc.at[pl.ds(zbase + r * ZR, ZR)], ssems[0])
        pltpu.async_copy(rows.at[0, pl.ds(0, 32)],
                         acc.at[pl.ds(zbase + 15 * ZR, 32)], ssems[0])
        for r in range(15):
            pltpu.make_async_copy(rows.at[0, pl.ds(0, ZR)],
                                  acc.at[pl.ds(zbase + r * ZR, ZR)],
                                  ssems[0]).wait()
        pltpu.make_async_copy(rows.at[0, pl.ds(0, 32)],
                              acc.at[pl.ds(zbase + 15 * ZR, 32)],
                              ssems[0]).wait()

    @pl.when(s == NS - 1)
    def _():
        for r in range(13):
            pltpu.async_copy(rows.at[0, pl.ds(0, ZR)],
                             acc.at[pl.ds(zbase + r * ZR, ZR)], ssems[0])
        for r in range(13):
            pltpu.make_async_copy(rows.at[0, pl.ds(0, ZR)],
                                  acc.at[pl.ds(zbase + r * ZR, ZR)],
                                  ssems[0]).wait()

    pltpu.make_async_copy(src_hbm.at[w, 0], sblk.at[0], isems[0]).wait()
    pltpu.make_async_copy(dst_hbm.at[w, 0], dblk.at[0], idsems[0]).wait()

    plsc.subcore_barrier()

    pltpu.async_copy(x_hbm.at[sblk.at[0, 0]], rows.at[0], gsems[0])

    def body(g, carry):
        for u in range(16):
            ru, rn = u % 2, (u + 1) % 2
            p, r = u // 8, u % 8

            pltpu.make_async_copy(x_hbm.at[sblk.at[p, r]], rows.at[ru],
                                  gsems[ru]).wait()
            pltpu.async_copy(rows.at[ru], acc.at[dblk.at[p, r]], ssems[ru],
                             add=True)

            pp, rr = ((u - 1) % 16) // 8, (u - 1) % 8
            if u == 0:
                @pl.when(g >= 1)
                def _():
                    pltpu.make_async_copy(rows.at[rn], acc.at[dblk.at[1, 7]],
                                          ssems[rn]).wait()
            else:
                pltpu.make_async_copy(rows.at[rn], acc.at[dblk.at[pp, rr]],
                                      ssems[rn]).wait()

            if u == 0:
                blk = 2 * g + 1
                pltpu.async_copy(src_hbm.at[w, blk], sblk.at[1], isems[1])
                pltpu.async_copy(dst_hbm.at[w, blk], dblk.at[1], idsems[1])
            if u == 8:
                @pl.when(g < NGRP - 1)
                def _():
                    blk = 2 * g + 2
                    pltpu.async_copy(src_hbm.at[w, blk], sblk.at[0], isems[0])
                    pltpu.async_copy(dst_hbm.at[w, blk], dblk.at[0],
                                     idsems[0])
            if u == 7:
                blk = 2 * g + 1
                pltpu.make_async_copy(src_hbm.at[w, blk], sblk.at[1],
                                      isems[1]).wait()
                pltpu.make_async_copy(dst_hbm.at[w, blk], dblk.at[1],
                                      idsems[1]).wait()
            if u == 15:
                @pl.when(g < NGRP - 1)
                def _():
                    blk = 2 * g + 2
                    pltpu.make_async_copy(src_hbm.at[w, blk], sblk.at[0],
                                          isems[0]).wait()
                    pltpu.make_async_copy(dst_hbm.at[w, blk], dblk.at[0],
                                          idsems[0]).wait()

            np_, nr = ((u + 1) % 16) // 8, (u + 1) % 8
            if u == 15:
                @pl.when(g < NGRP - 1)
                def _():
                    pltpu.async_copy(x_hbm.at[sblk.at[0, 0]], rows.at[rn],
                                     gsems[rn])
            else:
                pltpu.async_copy(x_hbm.at[sblk.at[np_, nr]], rows.at[rn],
                                 gsems[rn])
        return carry

    lax.fori_loop(0, NGRP, body, 0)

    pltpu.make_async_copy(rows.at[1], acc.at[dblk.at[1, 7]], ssems[1]).wait()

    plsc.subcore_barrier()

    @pl.when(s < NS - 1)
    def _():
        pltpu.sync_copy(acc.at[pl.ds(s * RPT, RPT)],
                        out_hbm.at[c, pl.ds(s * RPT, RPT)])

    @pl.when(s == NS - 1)
    def _():
        pltpu.sync_copy(acc.at[pl.ds((NS - 1) * RPT, RPT_LAST)],
                        out_hbm.at[c, pl.ds((NS - 1) * RPT, RPT_LAST)])


BN = 1000


def _tc_body(x_ref, p_ref, w_ref, b_ref, o_ref):
    h = x_ref[...] + p_ref[0] + p_ref[1]
    o_ref[...] = lax.dot_general(
        h, w_ref[...], (((1,), (1,)), ((), ())),
        preferred_element_type=jnp.float32) + b_ref[...]


def _tc_combine(x, parts, W, b2):
    return pl.pallas_call(
        _tc_body,
        grid=(N // BN,),
        in_specs=[
            pl.BlockSpec((BN, D), lambda i: (i, 0)),
            pl.BlockSpec((NC, BN, D), lambda i: (0, i, 0)),
            pl.BlockSpec((D, D), lambda i: (0, 0)),
            pl.BlockSpec((1, D), lambda i: (0, 0)),
        ],
        out_specs=pl.BlockSpec((BN, D), lambda i: (i, 0)),
        out_shape=jax.ShapeDtypeStruct((N, D), jnp.float32),
    )(x, parts, W, b2)


def kernel(x, edge_index, W, b):
    ei = edge_index.astype(jnp.int32).reshape(2, NW, NBLK, BLK, K)
    parts = _sc_aggregate(x, ei)
    return _tc_combine(x, parts, W, b.reshape(1, D))


# speedup vs baseline: 11.9741x; 1.0061x over previous
import functools

import jax
import jax.numpy as jnp
from jax import lax
from jax.experimental import pallas as pl
from jax.experimental.pallas import tpu as pltpu
from jax.experimental.pallas import tpu_sc as plsc

N = 10000
D = 128
E = 320000
NC = 2
NS = 16
NW = NC * NS
K = 125
NCHUNK = E // (NW * K)
BLK = 8
NBLK = NCHUNK // BLK
NGRP = NBLK // 2
RPT = 632
RPT_LAST = N - (NS - 1) * RPT
ZR = 40

KH1 = 64
KH2 = K - KH1

_mesh = plsc.VectorSubcoreMesh(core_axis_name="c", subcore_axis_name="s")


def _issue_gather(x_hbm, sblk, rows, sem, p, r, slot):
    pltpu.async_copy(x_hbm.at[sblk.at[p, r, pl.ds(0, KH1)]],
                     rows.at[slot, pl.ds(0, KH1)], sem)
    pltpu.async_copy(x_hbm.at[sblk.at[p, r, pl.ds(KH1, KH2)]],
                     rows.at[slot, pl.ds(KH1, KH2)], sem)


def _wait_gather(x_hbm, sblk, rows, sem, p, r, slot):
    pltpu.make_async_copy(x_hbm.at[sblk.at[p, r, pl.ds(0, KH1)]],
                          rows.at[slot, pl.ds(0, KH1)], sem).wait()
    pltpu.make_async_copy(x_hbm.at[sblk.at[p, r, pl.ds(KH1, KH2)]],
                          rows.at[slot, pl.ds(KH1, KH2)], sem).wait()


@functools.partial(
    pl.kernel,
    out_type=jax.ShapeDtypeStruct((NC, N, D), jnp.float32),
    mesh=_mesh,
    scratch_types=[
        pltpu.VMEM_SHARED((N, D), jnp.float32),
        pltpu.VMEM((2, BLK, K), jnp.int32),
        pltpu.VMEM((2, BLK, K), jnp.int32),
        pltpu.VMEM((2, K, D), jnp.float32),
        pltpu.SemaphoreType.DMA,
        pltpu.SemaphoreType.DMA,
        pltpu.SemaphoreType.DMA,
        pltpu.SemaphoreType.DMA,
        pltpu.SemaphoreType.DMA,
        pltpu.SemaphoreType.DMA,
        pltpu.SemaphoreType.DMA,
        pltpu.SemaphoreType.DMA,
    ],
)
def _sc_aggregate(x_hbm, ei_hbm, out_hbm,
                  acc, sblk, dblk, rows,
                  g0, g1, s0, s1, is0, is1, id0, id1):
    gsems = (g0, g1)
    ssems = (s0, s1)
    isems = (is0, is1)
    idsems = (id0, id1)
    c = lax.axis_index("c")
    s = lax.axis_index("s")
    w = c * NS + s
    src_hbm = ei_hbm.at[0]
    dst_hbm = ei_hbm.at[1]

    pltpu.async_copy(src_hbm.at[w, 0], sblk.at[0], isems[0])
    pltpu.async_copy(dst_hbm.at[w, 0], dblk.at[0], idsems[0])

    zv = jnp.zeros((16,), jnp.float32)
    for i in range(ZR):
        for jj in range(D // 16):
            rows[0, i, pl.ds(jj * 16, 16)] = zv

    zbase = s * RPT

    @pl.when(s < NS - 1)
    def _():
        for r in range(15):
            pltpu.async_copy(rows.at[0, pl.ds(0, ZR)],
                             acc.at[pl.ds(zbase + r * ZR, ZR)], ssems[0])
        pltpu.async_copy(rows.at[0, pl.ds(0, 32)],
                         acc.at[pl.ds(zbase + 15 * ZR, 32)], ssems[0])
        for r in range(15):
            pltpu.make_async_copy(rows.at[0, pl.ds(0, ZR)],
                                  acc.at[pl.ds(zbase + r * ZR, ZR)],
                                  ssems[0]).wait()
        pltpu.make_async_copy(rows.at[0, pl.ds(0, 32)],
                              acc.at[pl.ds(zbase + 15 * ZR, 32)],
                              ssems[0]).wait()

    @pl.when(s == NS - 1)
    def _():
        for r in range(13):
            pltpu.async_copy(rows.at[0, pl.ds(0, ZR)],
                             acc.at[pl.ds(zbase + r * ZR, ZR)], ssems[0])
        for r in range(13):
            pltpu.make_async_copy(rows.at[0, pl.ds(0, ZR)],
                                  acc.at[pl.ds(zbase + r * ZR, ZR)],
                                  ssems[0]).wait()

    pltpu.make_async_copy(src_hbm.at[w, 0], sblk.at[0], isems[0]).wait()
    pltpu.make_async_copy(dst_hbm.at[w, 0], dblk.at[0], idsems[0]).wait()

    plsc.subcore_barrier()

    _issue_gather(x_hbm, sblk, rows, gsems[0], 0, 0, 0)

    def body(g, carry):
        for u in range(16):
            ru, rn = u % 2, (u + 1) % 2
            p, r = u // 8, u % 8

            _wait_gather(x_hbm, sblk, rows, gsems[ru], p, r, ru)
            pltpu.async_copy(rows.at[ru], acc.at[dblk.at[p, r]], ssems[ru],
                             add=True)

            pp, rr = ((u - 1) % 16) // 8, (u - 1) % 8
            if u == 0:
                @pl.when(g >= 1)
                def _():
                    pltpu.make_async_copy(rows.at[rn], acc.at[dblk.at[1, 7]],
                                          ssems[rn]).wait()
            else:
                pltpu.make_async_copy(rows.at[rn], acc.at[dblk.at[pp, rr]],
                                      ssems[rn]).wait()

            if u == 0:
                blk = 2 * g + 1
                pltpu.async_copy(src_hbm.at[w, blk], sblk.at[1], isems[1])
                pltpu.async_copy(dst_hbm.at[w, blk], dblk.at[1], idsems[1])
            if u == 8:
                @pl.when(g < NGRP - 1)
                def _():
                    blk = 2 * g + 2
                    pltpu.async_copy(src_hbm.at[w, blk], sblk.at[0], isems[0])
                    pltpu.async_copy(dst_hbm.at[w, blk], dblk.at[0],
                                     idsems[0])
            if u == 7:
                blk = 2 * g + 1
                pltpu.make_async_copy(src_hbm.at[w, blk], sblk.at[1],
                                      isems[1]).wait()
                pltpu.make_async_copy(dst_hbm.at[w, blk], dblk.at[1],
                                      idsems[1]).wait()
            if u == 15:
                @pl.when(g < NGRP - 1)
                def _():
                    blk = 2 * g + 2
                    pltpu.make_async_copy(src_hbm.at[w, blk], sblk.at[0],
                                          isems[0]).wait()
                    pltpu.make_async_copy(dst_hbm.at[w, blk], dblk.at[0],
                                          idsems[0]).wait()

            np_, nr = ((u + 1) % 16) // 8, (u + 1) % 8
            if u == 15:
                @pl.when(g < NGRP - 1)
                def _():
                    _issue_gather(x_hbm, sblk, rows, gsems[rn], 0, 0, rn)
            else:
                _issue_gather(x_hbm, sblk, rows, gsems[rn], np_, nr, rn)
        return carry

    lax.fori_loop(0, NGRP, body, 0)

    pltpu.make_async_copy(rows.at[1], acc.at[dblk.at[1, 7]], ssems[1]).wait()

    plsc.subcore_barrier()

    @pl.when(s < NS - 1)
    def _():
        pltpu.sync_copy(acc.at[pl.ds(s * RPT, RPT)],
                        out_hbm.at[c, pl.ds(s * RPT, RPT)])

    @pl.when(s == NS - 1)
    def _():
        pltpu.sync_copy(acc.at[pl.ds((NS - 1) * RPT, RPT_LAST)],
                        out_hbm.at[c, pl.ds((NS - 1) * RPT, RPT_LAST)])


BN = 1000


def _tc_body(x_ref, p_ref, w_ref, b_ref, o_ref):
    h = x_ref[...] + p_ref[0] + p_ref[1]
    o_ref[...] = lax.dot_general(
        h, w_ref[...], (((1,), (1,)), ((), ())),
        preferred_element_type=jnp.float32) + b_ref[...]


def _tc_combine(x, parts, W, b2):
    return pl.pallas_call(
        _tc_body,
        grid=(N // BN,),
        in_specs=[
            pl.BlockSpec((BN, D), lambda i: (i, 0)),
            pl.BlockSpec((NC, BN, D), lambda i: (0, i, 0)),
            pl.BlockSpec((D, D), lambda i: (0, 0)),
            pl.BlockSpec((1, D), lambda i: (0, 0)),
        ],
        out_specs=pl.BlockSpec((BN, D), lambda i: (i, 0)),
        out_shape=jax.ShapeDtypeStruct((N, D), jnp.float32),
    )(x, parts, W, b2)


def kernel(x, edge_index, W, b):
    ei = edge_index.astype(jnp.int32).reshape(2, NW, NBLK, BLK, K)
    parts = _sc_aggregate(x, ei)
    return _tc_combine(x, parts, W, b.reshape(1, D))
